# baseline (speedup 1.0000x reference)
import functools

import jax
import jax.numpy as jnp
from jax import lax
from jax.experimental import pallas as pl
from jax.experimental.pallas import tpu as pltpu
from jax.experimental.pallas import tpu_sc as plsc

N_COLS = 26
VOCAB = 100000
B = 16384
D = 16

_INFO = plsc.get_sparse_core_info()
NW = _INFO.num_cores * _INFO.num_subcores
F_ROWS = N_COLS * D
F_PER_W = F_ROWS // NW
BQ = B // 4
UNROLL = 8
XCHUNKS = (0, 12544, 25088, 37632, 50176, 62720, 75264, 87808, VOCAB)
NXC = len(XCHUNKS) - 1


def _body(tbl_hbm, idx_hbm, out_hbm, xrow, idxbuf, y0, y1, xsem, ysem0, ysem1):
    wid = lax.axis_index("s") * _INFO.num_cores + lax.axis_index("c")
    zeros16 = jnp.zeros((16,), jnp.int32)
    ybufs = (y0, y1)
    ysems = (ysem0, ysem1)

    def ywrite(f, q, start):
        return
        cp = pltpu.make_async_copy(
            ybufs[q % 2], out_hbm.at[pl.ds(f, 1), pl.ds(q * BQ, BQ)],
            ysems[q % 2])
        if start:
            cp.start()
        else:
            cp.wait()

    def per_row(k, c_prev):
        f = wid * F_PER_W + k
        c = f // D

        copies = [
            pltpu.make_async_copy(
                tbl_hbm.at[pl.ds(f, 1), pl.ds(XCHUNKS[i], XCHUNKS[i + 1] - XCHUNKS[i])],
                xrow.at[:, pl.ds(XCHUNKS[i], XCHUNKS[i + 1] - XCHUNKS[i])],
                xsem,
            )
            for i in range(NXC)
        ]
        for cp in copies:
            cp.start()

        @pl.when(c != c_prev)
        def _():
            pltpu.sync_copy(idx_hbm.at[pl.ds(c, 1)], idxbuf)

        for cp in copies:
            cp.wait()

        for q in range(4):
            yq = ybufs[q % 2]
            if q >= 2:
                ywrite(f, q - 2, start=False)
            else:
                @pl.when(k > 0)
                def _():
                    ywrite(f - 1, q + 2, start=False)

            def grp(j, _, q=q, yq=yq):
                for u in range(UNROLL):
                    off = (j * UNROLL + u) * 16
                    v = idxbuf[0, pl.ds(q * BQ + off, 16)]
                    y = plsc.load_gather(xrow, [zeros16, v])
                    yq[0, pl.ds(off, 16)] = y
                return 0

            lax.fori_loop(0, BQ // (16 * UNROLL), grp, 0)
            ywrite(f, q, start=True)
        return c

    last_c = lax.fori_loop(0, F_PER_W, per_row, -1)
    f_last = wid * F_PER_W + F_PER_W - 1
    ywrite(f_last, 2, start=False)
    ywrite(f_last, 3, start=False)


def kernel(indices, tables):
    tbl = tables.transpose(0, 2, 1).reshape(F_ROWS, VOCAB)

    grid_kernel = pl.kernel(
        _body,
        out_type=jax.ShapeDtypeStruct((F_ROWS, B), jnp.float32),
        mesh=plsc.VectorSubcoreMesh(core_axis_name="c", subcore_axis_name="s"),
        scratch_types=[
            pltpu.VMEM((1, VOCAB), jnp.float32),
            pltpu.VMEM((1, B), jnp.int32),
            pltpu.VMEM((1, BQ), jnp.float32),
            pltpu.VMEM((1, BQ), jnp.float32),
            pltpu.SemaphoreType.DMA,
            pltpu.SemaphoreType.DMA,
            pltpu.SemaphoreType.DMA,
        ],
        compiler_params=pltpu.CompilerParams(
            use_tc_tiling_on_sc=True, needs_layout_passes=False),
    )
    out = grid_kernel(tbl, indices)
    return out.T


# speedup vs baseline: 1.3383x; 1.3383x over previous
import functools

import jax
import jax.numpy as jnp
from jax import lax
from jax.experimental import pallas as pl
from jax.experimental.pallas import tpu as pltpu
from jax.experimental.pallas import tpu_sc as plsc

N_COLS = 26
VOCAB = 100000
B = 16384
D = 16

_INFO = plsc.get_sparse_core_info()
NW = _INFO.num_cores * _INFO.num_subcores
F_ROWS = N_COLS * D
F_PER_W = F_ROWS // NW
BQ = B // 4
UNROLL = 8
XCHUNKS = (0, 12544, 25088, 37632, 50176, 62720, 75264, 87808, VOCAB)
NXC = len(XCHUNKS) - 1


def _body(tbl_hbm, idx_hbm, out_hbm, xrow, idxbuf, y0, y1, xsem, ysem0, ysem1):
    wid = lax.axis_index("s") * _INFO.num_cores + lax.axis_index("c")
    zeros16 = jnp.zeros((16,), jnp.int32)
    ybufs = (y0, y1)
    ysems = (ysem0, ysem1)

    def ywrite(f, q, start):
        cp = pltpu.make_async_copy(
            ybufs[q % 2], out_hbm.at[pl.ds(f, 1), pl.ds(q * BQ, BQ)],
            ysems[q % 2])
        if start:
            cp.start()
        else:
            cp.wait()

    def per_row(k, c_prev):
        f = wid * F_PER_W + k
        c = f // D

        copies = [
            pltpu.make_async_copy(
                tbl_hbm.at[pl.ds(f, 1), pl.ds(XCHUNKS[i], XCHUNKS[i + 1] - XCHUNKS[i])],
                xrow.at[:, pl.ds(XCHUNKS[i], XCHUNKS[i + 1] - XCHUNKS[i])],
                xsem,
            )
            for i in range(NXC)
        ]
        for cp in copies:
            cp.start()

        @pl.when(c != c_prev)
        def _():
            pltpu.sync_copy(idx_hbm.at[pl.ds(c, 1)], idxbuf)

        for cp in copies:
            cp.wait()

        for q in range(4):
            yq = ybufs[q % 2]
            if q >= 2:
                ywrite(f, q - 2, start=False)
            else:
                @pl.when(k > 0)
                def _():
                    ywrite(f - 1, q + 2, start=False)

            @plsc.parallel_loop(0, BQ // 16, unroll=UNROLL)
            def _(j, q=q, yq=yq):
                off = j * 16
                v = idxbuf[0, pl.ds(q * BQ + off, 16)]
                y = plsc.load_gather(xrow, [zeros16, v])
                yq[0, pl.ds(off, 16)] = y
            ywrite(f, q, start=True)
        return c

    last_c = lax.fori_loop(0, F_PER_W, per_row, -1)
    f_last = wid * F_PER_W + F_PER_W - 1
    ywrite(f_last, 2, start=False)
    ywrite(f_last, 3, start=False)


def kernel(indices, tables):
    tbl = tables.transpose(0, 2, 1).reshape(F_ROWS, VOCAB)

    grid_kernel = pl.kernel(
        _body,
        out_type=jax.ShapeDtypeStruct((F_ROWS, B), jnp.float32),
        mesh=plsc.VectorSubcoreMesh(core_axis_name="c", subcore_axis_name="s"),
        scratch_types=[
            pltpu.VMEM((1, VOCAB), jnp.float32),
            pltpu.VMEM((1, B), jnp.int32),
            pltpu.VMEM((1, BQ), jnp.float32),
            pltpu.VMEM((1, BQ), jnp.float32),
            pltpu.SemaphoreType.DMA,
            pltpu.SemaphoreType.DMA,
            pltpu.SemaphoreType.DMA,
        ],
        compiler_params=pltpu.CompilerParams(
            use_tc_tiling_on_sc=True, needs_layout_passes=False),
    )
    out = grid_kernel(tbl, indices)
    return out.T


# speedup vs baseline: 1.3424x; 1.0031x over previous
import jax
import jax.numpy as jnp
from jax import lax
from jax.experimental import pallas as pl
from jax.experimental.pallas import tpu as pltpu
from jax.experimental.pallas import tpu_sc as plsc

N_COLS = 26
VOCAB = 100000
B = 16384
D = 16

_INFO = plsc.get_sparse_core_info()
NW = _INFO.num_cores * _INFO.num_subcores
F_ROWS = N_COLS * D
F_PER_W = F_ROWS // NW
BQ = B // 4
UNROLL = 8
XCHUNKS = (0, 12544, 25088, 37632, 50176, 62720, 75264, 87808, VOCAB)
NXC = len(XCHUNKS) - 1


def _body(tbl_hbm, idx_hbm, out_hbm, xrow, idxbuf, y0, y1, xsem, ysem0, ysem1):
    wid = lax.axis_index("s") * _INFO.num_cores + lax.axis_index("c")
    zeros16 = jnp.zeros((16,), jnp.int32)
    ybufs = (y0, y1)
    ysems = (ysem0, ysem1)

    def ywrite(f, q, start):
        cp = pltpu.make_async_copy(
            ybufs[q % 2], out_hbm.at[pl.ds(f, 1), pl.ds(q * BQ, BQ)],
            ysems[q % 2])
        if start:
            cp.start()
        else:
            cp.wait()

    def per_row(k, c_prev):
        f = wid * F_PER_W + k
        c = f // D

        copies = [
            pltpu.make_async_copy(
                tbl_hbm.at[pl.ds(f, 1), pl.ds(XCHUNKS[i], XCHUNKS[i + 1] - XCHUNKS[i])],
                xrow.at[:, pl.ds(XCHUNKS[i], XCHUNKS[i + 1] - XCHUNKS[i])],
                xsem,
            )
            for i in range(NXC)
        ]
        for cp in copies:
            cp.start()

        @pl.when(c != c_prev)
        def _():
            pltpu.sync_copy(idx_hbm.at[pl.ds(c, 1)], idxbuf)

        for cp in copies:
            cp.wait()

        for q in range(4):
            yq = ybufs[q % 2]
            if q >= 2:
                ywrite(f, q - 2, start=False)
            else:
                @pl.when(k > 0)
                def _():
                    ywrite(f - 1, q + 2, start=False)

            @plsc.parallel_loop(0, BQ // 16, unroll=UNROLL)
            def _(j, q=q, yq=yq):
                off = j * 16
                v = idxbuf[0, pl.ds(q * BQ + off, 16)]
                y = plsc.load_gather(xrow, [zeros16, v])
                yq[0, pl.ds(off, 16)] = y
            ywrite(f, q, start=True)
        return c

    lax.fori_loop(0, F_PER_W, per_row, -1)
    f_last = wid * F_PER_W + F_PER_W - 1
    ywrite(f_last, 2, start=False)
    ywrite(f_last, 3, start=False)


def kernel(indices, tables):
    tbl = tables.transpose(0, 2, 1).reshape(F_ROWS, VOCAB)

    grid_kernel = pl.kernel(
        _body,
        out_type=jax.ShapeDtypeStruct((F_ROWS, B), jnp.float32),
        mesh=plsc.VectorSubcoreMesh(core_axis_name="c", subcore_axis_name="s"),
        scratch_types=[
            pltpu.VMEM((1, VOCAB), jnp.float32),
            pltpu.VMEM((1, B), jnp.int32),
            pltpu.VMEM((1, BQ), jnp.float32),
            pltpu.VMEM((1, BQ), jnp.float32),
            pltpu.SemaphoreType.DMA,
            pltpu.SemaphoreType.DMA,
            pltpu.SemaphoreType.DMA,
        ],
        compiler_params=pltpu.CompilerParams(
            use_tc_tiling_on_sc=True, needs_layout_passes=False),
    )
    out = grid_kernel(tbl, indices)
    return out.T
